# Initial kernel scaffold; baseline (speedup 1.0000x reference)
import jax
import jax.numpy as jnp
from jax.experimental import pallas as pl


def kernel(mask, pos_label, pred):
    raise NotImplementedError("write your pallas kernel here")


# speedup vs baseline: 3.2726x; 3.2726x over previous
import functools

import jax
import jax.numpy as jnp
import numpy as np
from jax.experimental import pallas as pl

HARD_NEG_MAX = 34.0
EASY_NEG_MAX = 15.0
POS_MAX = 18.0
LOGIT_THRESHOLD = float(np.log(np.float32(0.3)) - np.log(np.float32(0.7)))


def _label_body(mask_ref, pos_ref, pred_ref, out_ref):
    H, W = 384, 384
    m = mask_ref[0]
    p = pos_ref[0]
    x = pred_ref[0]

    k = jax.lax.broadcasted_iota(jnp.int32, (2 * W, W), 0)
    w = jax.lax.broadcasted_iota(jnp.int32, (2 * W, W), 1)
    sel = (k == 2 * w + 1).astype(jnp.float32) - (k == 2 * w).astype(jnp.float32)
    d = jnp.dot(x, sel, preferred_element_type=jnp.float32)

    score = (d >= LOGIT_THRESHOLD).astype(jnp.float32)
    neg = m * score

    i0 = jax.lax.broadcasted_iota(jnp.int32, (H, H), 0)
    i1 = jax.lax.broadcasted_iota(jnp.int32, (H, H), 1)
    upper_incl = (i0 <= i1).astype(jnp.float32)
    lower_strict = (i1 < i0).astype(jnp.float32)

    def rowmajor_cumsum(a):
        colcum = jnp.dot(a, upper_incl, preferred_element_type=jnp.float32)
        prev = jnp.dot(lower_strict, a, preferred_element_type=jnp.float32)
        prev_rows = jnp.sum(prev, axis=1, keepdims=True)
        return colcum + prev_rows

    keep_hard = neg * (rowmajor_cumsum(neg) <= HARD_NEG_MAX).astype(jnp.float32)
    keep_easy = m * (rowmajor_cumsum(m) <= EASY_NEG_MAX).astype(jnp.float32)
    keep_pos = p * (rowmajor_cumsum(p) <= POS_MAX).astype(jnp.float32)

    neg_f = (keep_hard + keep_easy >= 1.0).astype(jnp.float32)
    out_ref[0] = -1.0 + neg_f + 2.0 * keep_pos


def _run(mask, pos_label, pred, interpret=False):
    B, H, W = mask.shape
    pred2 = pred.reshape(B, H, 2 * W)
    return pl.pallas_call(
        _label_body,
        grid=(B,),
        in_specs=[
            pl.BlockSpec((1, H, W), lambda b: (b, 0, 0)),
            pl.BlockSpec((1, H, W), lambda b: (b, 0, 0)),
            pl.BlockSpec((1, H, 2 * W), lambda b: (b, 0, 0)),
        ],
        out_specs=pl.BlockSpec((1, H, W), lambda b: (b, 0, 0)),
        out_shape=jax.ShapeDtypeStruct((B, H, W), jnp.float32),
        interpret=interpret,
    )(mask, pos_label, pred2)


@jax.jit
def kernel(mask, pos_label, pred):
    return _run(mask, pos_label, pred)
